# baseline (speedup 1.0000x reference)
import functools

import jax
import jax.numpy as jnp
from jax import lax
from jax.experimental import pallas as pl
from jax.experimental.pallas import tpu as pltpu
from jax.experimental.pallas import tpu_sc as plsc

_B, _S, _D = 4096, 200, 64
_NV = 1000000
_N = _B * _S
_NC, _NS = 2, 16
_NW = _NC * _NS
_RPW = _N // _NW
_BLK = _S
_NBLK = _RPW // _BLK
_L = 16

_mesh = plsc.VectorSubcoreMesh(
    core_axis_name="c", subcore_axis_name="s", num_cores=_NC, num_subcores=_NS
)


@functools.partial(
    pl.kernel,
    out_type=jax.ShapeDtypeStruct((_N, _D), jnp.float32),
    mesh=_mesh,
    compiler_params=pltpu.CompilerParams(use_tc_tiling_on_sc=False),
    scratch_types=[
        pltpu.VMEM((_RPW,), jnp.int32),
        pltpu.VMEM((_S, _D), jnp.float32),
        pltpu.VMEM((_BLK, _D), jnp.float32),
        pltpu.SemaphoreType.DMA,
    ],
)
def _embed_kernel(x_hbm, table_hbm, pos_hbm, out_hbm, idx_v, pos_v, buf, gsem):
    wid = lax.axis_index("s") * _NC + lax.axis_index("c")
    base = wid * _RPW
    pltpu.sync_copy(pos_hbm, pos_v)
    pltpu.sync_copy(x_hbm.at[pl.ds(base, _RPW)], idx_v)

    def block(k, carry):
        pltpu.async_copy(
            table_hbm.at[idx_v.at[pl.ds(k * _BLK, _BLK)]], buf, gsem
        ).wait()

        def addrow(j, c):
            for c4 in range(_D // _L):
                sl = pl.ds(c4 * _L, _L)
                buf[j, sl] = buf[j, sl] + pos_v[j, sl]
            return c

        lax.fori_loop(0, _BLK, addrow, 0)
        pltpu.sync_copy(buf, out_hbm.at[pl.ds(base + k * _BLK, _BLK)])
        return carry

    lax.fori_loop(0, _NBLK, block, 0)


def kernel(x, token_table, position_embedding):
    xf = x.reshape(_N).astype(jnp.int32)
    out = _embed_kernel(xf, token_table, position_embedding)
    return out.reshape(_B, _S, _D)


# speedup vs baseline: 1.1483x; 1.1483x over previous
import functools

import jax
import jax.numpy as jnp
from jax import lax
from jax.experimental import pallas as pl
from jax.experimental.pallas import tpu as pltpu
from jax.experimental.pallas import tpu_sc as plsc

_B, _S, _D = 4096, 200, 64
_NV = 1000000
_N = _B * _S
_NC, _NS = 2, 16
_NW = _NC * _NS
_RPW = _N // _NW
_BLK = _S
_NBLK = _RPW // _BLK
_L = 16

_mesh = plsc.VectorSubcoreMesh(
    core_axis_name="c", subcore_axis_name="s", num_cores=_NC, num_subcores=_NS
)


_NBUF = 4


@functools.partial(
    pl.kernel,
    out_type=jax.ShapeDtypeStruct((_N, _D), jnp.float32),
    mesh=_mesh,
    compiler_params=pltpu.CompilerParams(use_tc_tiling_on_sc=False),
    scratch_types=[
        pltpu.VMEM((_RPW,), jnp.int32),
        pltpu.VMEM((_S, _D), jnp.float32),
        [pltpu.VMEM((_BLK, _D), jnp.float32) for _ in range(_NBUF)],
        [pltpu.SemaphoreType.DMA for _ in range(_NBUF)],
        [pltpu.SemaphoreType.DMA for _ in range(_NBUF)],
    ],
)
def _embed_kernel(x_hbm, table_hbm, pos_hbm, out_hbm, idx_v, pos_v, bufs, gs, ws):
    wid = lax.axis_index("s") * _NC + lax.axis_index("c")
    base = wid * _RPW
    pltpu.sync_copy(pos_hbm, pos_v)
    pltpu.sync_copy(x_hbm.at[pl.ds(base, _RPW)], idx_v)

    def start_gather(k, b):
        pltpu.async_copy(
            table_hbm.at[idx_v.at[pl.ds(k * _BLK, _BLK)]], bufs[b], gs[b]
        )

    def wait_gather(b):
        pltpu.make_async_copy(out_hbm.at[pl.ds(0, _BLK)], bufs[b], gs[b]).wait()

    def start_write(k, b):
        pltpu.async_copy(bufs[b], out_hbm.at[pl.ds(base + k * _BLK, _BLK)], ws[b])

    def wait_write(b):
        pltpu.make_async_copy(bufs[b], out_hbm.at[pl.ds(0, _BLK)], ws[b]).wait()

    def add_pos(b):
        buf = bufs[b]

        @plsc.parallel_loop(0, _BLK, unroll=4)
        def _(j):
            for c4 in range(_D // _L):
                sl = pl.ds(c4 * _L, _L)
                buf[j, sl] = buf[j, sl] + pos_v[j, sl]

    for b in range(_NBUF):
        start_gather(b, b)

    def outer(i, carry):
        for b in range(_NBUF):
            k = i * _NBUF + b
            wait_gather(b)
            add_pos(b)
            start_write(k, b)
            j = k - 2
            bb = (b + 2) % _NBUF

            @pl.when(jnp.logical_and(j >= 0, j + _NBUF < _NBLK))
            def _():
                wait_write(bb)
                start_gather(j + _NBUF, bb)

        return carry

    lax.fori_loop(0, _NBLK // _NBUF, outer, 0)
    for b in range(_NBUF):
        wait_write(b)


def kernel(x, token_table, position_embedding):
    xf = x.reshape(_N).astype(jnp.int32)
    out = _embed_kernel(xf, token_table, position_embedding)
    return out.reshape(_B, _S, _D)
